# baseline (speedup 1.0000x reference)
import functools

import jax
import jax.numpy as jnp
from jax import lax
from jax.experimental import pallas as pl
from jax.experimental.pallas import tpu as pltpu
from jax.experimental.pallas import tpu_sc as plsc

MAXLEN = 200
EMBED = 64
PADE = 128
NC = 2
NS = 16
NW = NC * NS
LANES = 16
NG = 2
NO = 4


def _body(xt_hbm, tok_hbm, pos_hbm, out_hbm, idx_v, pos_v, rows_v, rows_t,
          g0, g1, o0, o1, o2, o3):
    gsems = (g0, g1)
    osems = (o0, o1, o2, o3)
    cid = lax.axis_index("c")
    sid = lax.axis_index("s")
    wid = sid * NC + cid
    nb = xt_hbm.shape[1] // NW
    b0 = wid * nb
    nchunk = xt_hbm.shape[0]
    nblock = nchunk // NO

    pltpu.sync_copy(xt_hbm.at[:, pl.ds(b0, nb)], idx_v)
    pltpu.sync_copy(pos_hbm, pos_v)

    iota = lax.iota(jnp.int32, LANES)
    etv = [lax.shift_right_logical(iota + j * LANES, 3)
           for j in range(EMBED // LANES)]
    einv = [lax.shift_left(lax.bitwise_and(iota + j * LANES, 7), 7)
            for j in range(EMBED // LANES)]

    def gather_start(l, bg):
        pltpu.async_copy(tok_hbm.at[idx_v.at[l]], rows_v.at[bg], gsems[bg])

    def gather_wait(l, bg):
        pltpu.make_async_copy(
            tok_hbm.at[idx_v.at[l]], rows_v.at[bg], gsems[bg]).wait()

    def write_start(l, bo):
        pltpu.async_copy(
            rows_t.at[bo], out_hbm.at[l, :, pl.ds(wid * 1024, 1024)],
            osems[bo])

    def write_wait(l, bo):
        pltpu.make_async_copy(
            rows_t.at[bo], out_hbm.at[l, :, pl.ds(wid * 1024, 1024)],
            osems[bo]).wait()

    def compute(l, bg, bo):
        pv = [pos_v[l, pl.ds(j * LANES, LANES)] for j in range(EMBED // LANES)]

        def r_body(r4, _):
            for dr in range(4):
                r = r4 * 4 + dr
                for j in range(EMBED // LANES):
                    vals = rows_v[bg, r, pl.ds(j * LANES, LANES)] + pv[j]
                    plsc.store_scatter(rows_t.at[bo], [etv[j], einv[j] + r],
                                       vals)
            return 0

        lax.fori_loop(0, nb // 4, r_body, 0)

    def step(l, bg, bo, first, last):
        gather_wait(l, bg)
        if (not last) or l + 1 < nchunk:
            gather_start(l + 1, (bg + 1) % NG)
        if (not first) or l >= NO:
            write_wait(l - NO, bo)
        compute(l, bg, bo)
        write_start(l, bo)

    gather_start(0, 0)

    for l in range(NO):
        step(l, l % NG, l, True, False)

    def block(k, _):
        l0 = k * NO
        for b in range(NO):
            step(l0 + b, b % NG, b, False, False)
        return 0

    lax.fori_loop(1, nblock - 1, block, 0)

    l0 = (nblock - 1) * NO
    for b in range(NO):
        step(l0 + b, b % NG, b, False, True)

    for l in range(nchunk - NO, nchunk):
        write_wait(l, l % NO)


def kernel(x, tok_table, pos_table):
    B, L = x.shape
    assert B % NW == 0 and L % NO == 0
    nb = B // NW

    tok128 = jnp.pad(tok_table, ((0, 0), (0, PADE - EMBED)))

    mesh = plsc.VectorSubcoreMesh(core_axis_name="c", subcore_axis_name="s")
    run = functools.partial(
        pl.kernel,
        mesh=mesh,
        compiler_params=pltpu.CompilerParams(use_tc_tiling_on_sc=False,
                                             needs_layout_passes=False),
        out_type=jax.ShapeDtypeStruct((L, EMBED // 8, B * 8), jnp.float32),
        scratch_types=[
            pltpu.VMEM((L, nb), jnp.int32),
            pltpu.VMEM((L, EMBED), jnp.float32),
            pltpu.VMEM((NG, nb, PADE), jnp.float32),
            pltpu.VMEM((NO, EMBED // 8, 8 * nb), jnp.float32),
        ] + [pltpu.SemaphoreType.DMA] * (NG + NO),
    )(_body)
    out = run(x.T, tok128, pos_table)
    out = jax.lax.optimization_barrier(out)
    out5 = out.reshape(L, EMBED // 8, B // 128, 8, 128)
    return out5.transpose(2, 4, 0, 1, 3).reshape(B, L, EMBED)
